# baseline (speedup 1.0000x reference)
import functools

import jax
import jax.numpy as jnp
from jax import lax
from jax.experimental import pallas as pl
from jax.experimental.pallas import tpu as pltpu
from jax.experimental.pallas import tpu_sc as plsc

N = 10000
NNZ = 320000
D = 128

NC = 2
NS = 16
NW = NC * NS
CHUNK = 64
NBUF = 4
CPW_A = 320
CPW_B = 160
STG = 16
NNZ_PAD = NS * CPW_A * CHUNK
ROWS1 = NNZ_PAD // CHUNK
TBL = 40400
X2OFF = 2 * N
ZROW = 4 * N
ZCH = 1000


def _prep_cat(X0, X2):
    def body(x0_ref, x2_ref, o_ref):
        g = pl.program_id(0)
        c0 = jnp.where(g < 25, 1.0, jnp.where(g < 50, -1.0, 0.0))
        c2 = jnp.where(g < 50, 0.0,
                       jnp.where(g < 75, 1.0, jnp.where(g < 100, -1.0, 0.0)))
        o_ref[...] = c0 * x0_ref[...] + c2 * x2_ref[...]

    return pl.pallas_call(
        body,
        grid=(TBL // 400,),
        in_specs=[pl.BlockSpec((400, D), lambda g: (g % 25, 0))] * 2,
        out_specs=pl.BlockSpec((400, D), lambda g: (g, 0)),
        out_shape=jax.ShapeDtypeStruct((TBL, D), jnp.float32),
    )(X0, X2)


def _prep_idx(ca, va, cb, vb):
    def body(car, var, cbr, vbr, oar, obr):
        off = jnp.int32(N)
        zero = jnp.int32(0)
        oar[...] = car[...] + jnp.where(var[...] < 0.0, off, zero)
        obr[...] = cbr[...] + jnp.where(vbr[...] < 0.0, off, zero)

    ra, rb = 2 * ROWS1, ROWS1
    return pl.pallas_call(
        body,
        grid=(4,),
        in_specs=[
            pl.BlockSpec((ra // 4, CHUNK), lambda g: (g, 0)),
            pl.BlockSpec((ra // 4, CHUNK), lambda g: (g, 0)),
            pl.BlockSpec((rb // 4, CHUNK), lambda g: (g, 0)),
            pl.BlockSpec((rb // 4, CHUNK), lambda g: (g, 0)),
        ],
        out_specs=[
            pl.BlockSpec((ra // 4, CHUNK), lambda g: (g, 0)),
            pl.BlockSpec((rb // 4, CHUNK), lambda g: (g, 0)),
        ],
        out_shape=[
            jax.ShapeDtypeStruct((ra, CHUNK), jnp.int32),
            jax.ShapeDtypeStruct((rb, CHUNK), jnp.int32),
        ],
    )(ca, va, cb, vb)


def _tc_mm(X1, X2, W1, W2):
    def body(x1_ref, x2_ref, w1_ref, w2_ref, y1_ref, x2o_ref):
        dn = (((1,), (1,)), ((), ()))
        y1_ref[...] = lax.dot_general(x1_ref[...], w1_ref[...], dn,
                                      preferred_element_type=jnp.float32)
        z = lax.dot_general(x2_ref[...], w2_ref[...], dn,
                            preferred_element_type=jnp.float32)
        x2o_ref[...] = z * jax.nn.sigmoid(z)

    return pl.pallas_call(
        body,
        grid=(10,),
        in_specs=[
            pl.BlockSpec((1000, D), lambda g: (g, 0)),
            pl.BlockSpec((1000, D), lambda g: (g, 0)),
            pl.BlockSpec((D, D), lambda g: (0, 0)),
            pl.BlockSpec((D, D), lambda g: (0, 0)),
        ],
        out_specs=[pl.BlockSpec((1000, D), lambda g: (g, 0))] * 2,
        out_shape=[jax.ShapeDtypeStruct((N, D), jnp.float32)] * 2,
    )(X1, X2, W1, W2)


def _sc_spmm(tab, ga, ra, gb, rb, zeros):
    mesh = plsc.VectorSubcoreMesh(core_axis_name="c", subcore_axis_name="s")

    @functools.partial(
        pl.kernel,
        mesh=mesh,
        out_type=[jax.ShapeDtypeStruct((2 * N, D), jnp.float32)] * 2,
        scratch_types=[
            pltpu.VMEM((STG, CHUNK), jnp.int32),
            pltpu.VMEM((STG, CHUNK), jnp.int32),
            pltpu.VMEM((NBUF, CHUNK, D), jnp.float32),
            pltpu.VMEM_SHARED((N, D), jnp.float32),
            pltpu.SemaphoreType.DMA((NBUF,)),
            pltpu.SemaphoreType.DMA((NBUF,)),
        ],
    )
    def k(tab_h, ga_h, ra_h, gb_h, rb_h, z_h, oa_h, ob_h,
          gidx, ridx, bufs, acc, semg, sems):
        c = lax.axis_index("c")
        s = lax.axis_index("s")
        wid = c * NS + s
        zb = s * ZCH
        nr = STG // NBUF

        for g_h, r_h, out_h, cpw in (
            (ga_h, ra_h, oa_h, CPW_A),
            (gb_h, rb_h, ob_h, CPW_B),
        ):
            rowbase = wid * cpw
            @pl.when(s < N // ZCH)
            def _():
                pltpu.sync_copy(z_h.at[pl.ds(zb, ZCH)], acc.at[pl.ds(zb, ZCH)])
            plsc.subcore_barrier()

            for st in range(cpw // STG):
                base = rowbase + st * STG
                pltpu.sync_copy(g_h.at[pl.ds(base, STG)], gidx)
                pltpu.sync_copy(r_h.at[pl.ds(base, STG)], ridx)

                for b in range(NBUF):
                    pltpu.async_copy(
                        tab_h.at[gidx.at[b]], bufs.at[b], semg.at[b])

                def rnd(r, carry):
                    for b in range(NBUF):
                        j = r * NBUF + b
                        pltpu.make_async_copy(
                            tab_h.at[gidx.at[j]], bufs.at[b],
                            semg.at[b]).wait()
                        pltpu.async_copy(
                            bufs.at[b], acc.at[ridx.at[j]], sems.at[b],
                            add=True)

                    @pl.when(r < nr - 1)
                    def _():
                        for b in range(NBUF):
                            j = r * NBUF + b
                            pltpu.make_async_copy(
                                bufs.at[b], acc.at[ridx.at[j]],
                                sems.at[b]).wait()
                            pltpu.async_copy(
                                tab_h.at[gidx.at[j + NBUF]], bufs.at[b],
                                semg.at[b])

                    return carry

                lax.fori_loop(0, nr, rnd, 0)

                for b in range(NBUF):
                    pltpu.make_async_copy(
                        bufs.at[b], acc.at[ridx.at[STG - NBUF + b]],
                        sems.at[b]).wait()

            plsc.subcore_barrier()

            @pl.when(s < N // ZCH)
            def _():
                pltpu.sync_copy(acc.at[pl.ds(zb, ZCH)],
                                out_h.at[pl.ds(c * N + zb, ZCH)])

            plsc.subcore_barrier()

    return k(tab, ga, ra, gb, rb, zeros)


def _tc_combine(s1, s2, s3a, s3b, y1, W0, alpha1):
    def body(s1_r, s2_r, s3a_r, s3b_r, y1_r, w0_r, al_r, x0o_r, x1o_r):
        a = al_r[0]
        dn = (((1,), (1,)), ((), ()))
        t = lax.dot_general(s1_r[...], w0_r[...], dn,
                            preferred_element_type=jnp.float32)
        z0 = a * t + (1.0 - a) * s2_r[...]
        x0o_r[...] = z0 * jax.nn.sigmoid(z0)
        z1 = 0.5 * (y1_r[...] + s3a_r[...] + s3b_r[...])
        x1o_r[...] = z1 * jax.nn.sigmoid(z1)

    blk = pl.BlockSpec((1000, D), lambda g: (g, 0))
    return pl.pallas_call(
        body,
        grid=(10,),
        in_specs=[blk] * 5 + [
            pl.BlockSpec((D, D), lambda g: (0, 0)),
            pl.BlockSpec(memory_space=pltpu.SMEM),
        ],
        out_specs=[blk] * 2,
        out_shape=[jax.ShapeDtypeStruct((N, D), jnp.float32)] * 2,
    )(s1, s2, s3a, s3b, y1, W0, alpha1)


def kernel(X0, X1, X2, B1_rows, B1_cols, B1_vals,
           B2_rows, B2_cols, B2_vals, W0, W1, W2, alpha):
    padn = NNZ_PAD - NNZ
    padi = jnp.arange(padn, dtype=jnp.int32)
    padg = ZROW + padi % (TBL - ZROW)
    padr = padi % N
    padv = jnp.ones((padn,), jnp.float32)

    def cat2d(a, pad):
        return jnp.concatenate([a, pad]).reshape(-1, CHUNK)

    ca = jnp.concatenate([cat2d(B1_cols, padg),
                          cat2d(B2_cols + X2OFF, padg)])
    va = jnp.concatenate([cat2d(B1_vals, padv), cat2d(B2_vals, padv)])
    rA = jnp.concatenate([cat2d(B1_rows, padr), cat2d(B2_rows, padr)])
    cb = cat2d(B2_rows + X2OFF, padg)
    rB = cat2d(B2_cols, padr)
    vb = cat2d(B2_vals, padv)

    tab = _prep_cat(X0, X2)
    gA, gB = _prep_idx(ca, va, cb, vb)
    y1, x2_out = _tc_mm(X1, X2, W1, W2)

    zeros = jnp.zeros((N, D), jnp.float32)
    sAp, sBp = _sc_spmm(tab, gA, rA, gB, rB, zeros)

    x0_out, x1_out = _tc_combine(
        sAp[:N], sAp[N:], sBp[:N], sBp[N:], y1, W0, alpha.reshape(1))
    return (x0_out, x1_out, x2_out)


# speedup vs baseline: 1.1351x; 1.1351x over previous
import functools

import jax
import jax.numpy as jnp
from jax import lax
from jax.experimental import pallas as pl
from jax.experimental.pallas import tpu as pltpu
from jax.experimental.pallas import tpu_sc as plsc

N = 10000
NNZ = 320000
D = 128

NC = 2
NS = 16
NW = NC * NS
CHUNK = 64
NBUF = 4
CPW = 160
STG = 40
NNZ_PAD = NW * CPW * CHUNK
IDXROWS = NNZ_PAD // CHUNK
TBL = 20400
ZROW = 2 * N
ZCH = 1000


def _prep_cat(X0, X2):
    def body(x0_ref, x2_ref, o0_ref, o2_ref):
        g = pl.program_id(0)
        coef = jnp.where(g < 25, 1.0, jnp.where(g < 50, -1.0, 0.0))
        o0_ref[...] = coef * x0_ref[...]
        o2_ref[...] = coef * x2_ref[...]

    return pl.pallas_call(
        body,
        grid=(TBL // 400,),
        in_specs=[pl.BlockSpec((400, D), lambda g: (g % 25, 0))] * 2,
        out_specs=[pl.BlockSpec((400, D), lambda g: (g, 0))] * 2,
        out_shape=[jax.ShapeDtypeStruct((TBL, D), jnp.float32)] * 2,
    )(X0, X2)


def _prep_idx(c1, v1, c2, v2, c3, v3):
    def body(c1r, v1r, c2r, v2r, c3r, v3r, o1r, o2r, o3r):
        off = jnp.int32(N)
        zero = jnp.int32(0)
        o1r[...] = c1r[...] + jnp.where(v1r[...] < 0.0, off, zero)
        o2r[...] = c2r[...] + jnp.where(v2r[...] < 0.0, off, zero)
        o3r[...] = c3r[...] + jnp.where(v3r[...] < 0.0, off, zero)

    blk = IDXROWS // 4
    return pl.pallas_call(
        body,
        grid=(4,),
        in_specs=[pl.BlockSpec((blk, CHUNK), lambda g: (g, 0))] * 6,
        out_specs=[pl.BlockSpec((blk, CHUNK), lambda g: (g, 0))] * 3,
        out_shape=[jax.ShapeDtypeStruct((IDXROWS, CHUNK), jnp.int32)] * 3,
    )(c1, v1, c2, v2, c3, v3)


def _tc_mm(X1, X2, W1, W2):
    def body(x1_ref, x2_ref, w1_ref, w2_ref, y1_ref, x2o_ref):
        dn = (((1,), (1,)), ((), ()))
        y1_ref[...] = lax.dot_general(x1_ref[...], w1_ref[...], dn,
                                      preferred_element_type=jnp.float32)
        z = lax.dot_general(x2_ref[...], w2_ref[...], dn,
                            preferred_element_type=jnp.float32)
        x2o_ref[...] = z * jax.nn.sigmoid(z)

    return pl.pallas_call(
        body,
        grid=(10,),
        in_specs=[
            pl.BlockSpec((1000, D), lambda g: (g, 0)),
            pl.BlockSpec((1000, D), lambda g: (g, 0)),
            pl.BlockSpec((D, D), lambda g: (0, 0)),
            pl.BlockSpec((D, D), lambda g: (0, 0)),
        ],
        out_specs=[pl.BlockSpec((1000, D), lambda g: (g, 0))] * 2,
        out_shape=[jax.ShapeDtypeStruct((N, D), jnp.float32)] * 2,
    )(X1, X2, W1, W2)


def _sc_spmm(x0cat, x2cat, g1, r1, g2, r2, g3, r3, zeros):
    mesh = plsc.VectorSubcoreMesh(core_axis_name="c", subcore_axis_name="s")

    @functools.partial(
        pl.kernel,
        mesh=mesh,
        out_type=[jax.ShapeDtypeStruct((2 * N, D), jnp.float32)] * 3,
        scratch_types=[
            pltpu.VMEM((STG, CHUNK), jnp.int32),
            pltpu.VMEM((STG, CHUNK), jnp.int32),
            pltpu.VMEM((NBUF, CHUNK, D), jnp.float32),
            pltpu.VMEM_SHARED((N, D), jnp.float32),
            pltpu.SemaphoreType.DMA((NBUF,)),
            pltpu.SemaphoreType.DMA((NBUF,)),
        ],
    )
    def k(x0c_h, x2c_h, g1_h, r1_h, g2_h, r2_h, g3_h, r3_h, z_h,
          s1_h, s2_h, s3_h, gidx, ridx, bufs, acc, semg, sems):
        c = lax.axis_index("c")
        s = lax.axis_index("s")
        wid = c * NS + s
        rowbase = wid * CPW
        zb = s * ZCH
        nr = STG // NBUF

        for tab_h, g_h, r_h, out_h in (
            (x0c_h, g1_h, r1_h, s1_h),
            (x2c_h, g2_h, r2_h, s2_h),
            (x2c_h, g3_h, r3_h, s3_h),
        ):
            @pl.when(s < N // ZCH)
            def _():
                pltpu.sync_copy(z_h.at[pl.ds(zb, ZCH)], acc.at[pl.ds(zb, ZCH)])
            plsc.subcore_barrier()

            for st in range(CPW // STG):
                base = rowbase + st * STG
                pltpu.sync_copy(g_h.at[pl.ds(base, STG)], gidx)
                pltpu.sync_copy(r_h.at[pl.ds(base, STG)], ridx)

                for b in range(NBUF):
                    pltpu.async_copy(
                        tab_h.at[gidx.at[b]], bufs.at[b], semg.at[b])

                def rnd(r, carry):
                    for b in range(NBUF):
                        j = r * NBUF + b
                        pltpu.make_async_copy(
                            tab_h.at[gidx.at[j]], bufs.at[b],
                            semg.at[b]).wait()
                        pltpu.async_copy(
                            bufs.at[b], acc.at[ridx.at[j]], sems.at[b],
                            add=True)

                    @pl.when(r < nr - 1)
                    def _():
                        for b in range(NBUF):
                            j = r * NBUF + b
                            pltpu.make_async_copy(
                                bufs.at[b], acc.at[ridx.at[j]],
                                sems.at[b]).wait()
                            pltpu.async_copy(
                                tab_h.at[gidx.at[j + NBUF]], bufs.at[b],
                                semg.at[b])

                    return carry

                lax.fori_loop(0, nr, rnd, 0)

                for b in range(NBUF):
                    pltpu.make_async_copy(
                        bufs.at[b], acc.at[ridx.at[STG - NBUF + b]],
                        sems.at[b]).wait()

            plsc.subcore_barrier()

            @pl.when(s < N // ZCH)
            def _():
                pltpu.sync_copy(acc.at[pl.ds(zb, ZCH)],
                                out_h.at[pl.ds(c * N + zb, ZCH)])

            plsc.subcore_barrier()

    return k(x0cat, x2cat, g1, r1, g2, r2, g3, r3, zeros)


def _tc_combine(s1a, s1b, s2a, s2b, s3a, s3b, y1, W0, alpha1):
    def body(s1a_r, s1b_r, s2a_r, s2b_r, s3a_r, s3b_r, y1_r, w0_r, al_r,
             x0o_r, x1o_r):
        a = al_r[0]
        dn = (((1,), (1,)), ((), ()))
        s1 = s1a_r[...] + s1b_r[...]
        t = lax.dot_general(s1, w0_r[...], dn,
                            preferred_element_type=jnp.float32)
        z0 = a * t + (1.0 - a) * (s2a_r[...] + s2b_r[...])
        x0o_r[...] = z0 * jax.nn.sigmoid(z0)
        z1 = 0.5 * (y1_r[...] + s3a_r[...] + s3b_r[...])
        x1o_r[...] = z1 * jax.nn.sigmoid(z1)

    blk = pl.BlockSpec((1000, D), lambda g: (g, 0))
    return pl.pallas_call(
        body,
        grid=(10,),
        in_specs=[blk] * 7 + [
            pl.BlockSpec((D, D), lambda g: (0, 0)),
            pl.BlockSpec(memory_space=pltpu.SMEM),
        ],
        out_specs=[blk] * 2,
        out_shape=[jax.ShapeDtypeStruct((N, D), jnp.float32)] * 2,
    )(s1a, s1b, s2a, s2b, s3a, s3b, y1, W0, alpha1)


def kernel(X0, X1, X2, B1_rows, B1_cols, B1_vals,
           B2_rows, B2_cols, B2_vals, W0, W1, W2, alpha):
    padn = NNZ_PAD - NNZ
    padi = jnp.arange(padn, dtype=jnp.int32)
    padg = ZROW + padi % (TBL - ZROW)
    padr = padi % N
    padv = jnp.ones((padn,), jnp.float32)

    def cat2d(a, pad):
        return jnp.concatenate([a, pad]).reshape(IDXROWS, CHUNK)

    c1 = cat2d(B1_cols, padg)
    v1 = cat2d(B1_vals, padv)
    r1 = cat2d(B1_rows, padr)
    c2 = cat2d(B2_cols, padg)
    v2 = cat2d(B2_vals, padv)
    r2 = cat2d(B2_rows, padr)
    c3 = cat2d(B2_rows, padg)
    r3 = cat2d(B2_cols, padr)

    x0cat, x2cat = _prep_cat(X0, X2)
    g1, g2, g3 = _prep_idx(c1, v1, c2, v2, c3, v2)
    y1, x2_out = _tc_mm(X1, X2, W1, W2)

    zeros = jnp.zeros((N, D), jnp.float32)
    s1p, s2p, s3p = _sc_spmm(x0cat, x2cat, g1, r1, g2, r2, g3, r3, zeros)

    x0_out, x1_out = _tc_combine(
        s1p[:N], s1p[N:], s2p[:N], s2p[N:], s3p[:N], s3p[N:],
        y1, W0, alpha.reshape(1))
    return (x0_out, x1_out, x2_out)
